# baseline (speedup 1.0000x reference)
import functools

import jax
import jax.numpy as jnp
from jax.experimental import pallas as pl
from jax.experimental.pallas import tpu as pltpu

_N_C, _N_L, _D = 10000, 4096, 128
_K_BLK = 1000
_K_STEPS = _N_C // _K_BLK
_M_BLK = _N_L // 2


def _fused_body(adj_ref, xc_ref, xl_ref, xlf_ref, c0_ref, wmsg_ref, wflip_ref,
                whh_ref, bias_ref, h_ref, c_ref, acc_ref):
    k = pl.program_id(1)

    @pl.when(k == 0)
    def _init():
        acc_ref[...] = jnp.zeros_like(acc_ref)

    acc_ref[...] += jax.lax.dot_general(
        adj_ref[...], xc_ref[...],
        dimension_numbers=(((0,), (0,)), ((), ())),
        preferred_element_type=jnp.float32)

    @pl.when(k == _K_STEPS - 1)
    def _finish():
        def mm(a, b):
            return jax.lax.dot_general(
                a, b, dimension_numbers=(((1,), (0,)), ((), ())),
                preferred_element_type=jnp.float32)

        gates = (mm(acc_ref[...], wmsg_ref[...]) + mm(xlf_ref[...], wflip_ref[...])
                 + mm(xl_ref[...], whh_ref[...]) + bias_ref[...])
        i = jax.nn.sigmoid(gates[:, :_D])
        f = jax.nn.sigmoid(gates[:, _D:2 * _D])
        g = jnp.tanh(gates[:, 2 * _D:3 * _D])
        o = jax.nn.sigmoid(gates[:, 3 * _D:])
        c = f * c0_ref[...] + i * g
        h_ref[...] = o * jnp.tanh(c)
        c_ref[...] = c


@functools.partial(jax.jit, static_argnames=())
def kernel(adj_t, x_c, hidden, l_batch, W_ih, W_hh, b_ih, b_hh):
    del l_batch
    x_l = hidden[0]
    c0 = hidden[1]
    wih_t = W_ih.T
    w_msg = wih_t[:_D]
    w_flip = wih_t[_D:]
    whh_t = W_hh.T
    bias = (b_ih + b_hh)[None, :]

    const = lambda shape: pl.BlockSpec(shape, lambda m, k: (0, 0))
    h, c = pl.pallas_call(
        _fused_body,
        grid=(2, _K_STEPS),
        in_specs=[
            pl.BlockSpec((_K_BLK, _M_BLK), lambda m, k: (k, m)),
            pl.BlockSpec((_K_BLK, _D), lambda m, k: (k, 0)),
            pl.BlockSpec((_M_BLK, _D), lambda m, k: (m, 0)),
            pl.BlockSpec((_M_BLK, _D), lambda m, k: (1 - m, 0)),
            pl.BlockSpec((_M_BLK, _D), lambda m, k: (m, 0)),
            const((_D, 4 * _D)),
            const((_D, 4 * _D)),
            const((_D, 4 * _D)),
            const((1, 4 * _D)),
        ],
        out_specs=[pl.BlockSpec((_M_BLK, _D), lambda m, k: (m, 0))] * 2,
        out_shape=[jax.ShapeDtypeStruct((_N_L, _D), jnp.float32)] * 2,
        scratch_shapes=[pltpu.VMEM((_M_BLK, _D), jnp.float32)],
        compiler_params=pltpu.CompilerParams(
            dimension_semantics=("parallel", "arbitrary")),
    )(adj_t, x_c, x_l, x_l, c0, w_msg, w_flip, whh_t, bias)
    return (h, c)


# speedup vs baseline: 1.0522x; 1.0522x over previous
import functools

import jax
import jax.numpy as jnp
from jax.experimental import pallas as pl
from jax.experimental.pallas import tpu as pltpu

_N_C, _N_L, _D = 10000, 4096, 128
_CHUNK = 400
_N_CHUNKS = _N_C // _CHUNK
_N_BUF = 4


def _fused_body(adj_ref, xc_ref, xl_ref, c0_ref, wmsg_ref, wflip_ref,
                whh_ref, bias_ref, h_ref, c_ref, bufs_ref, acc_ref,
                gpart_ref, sems_ref):
    def start(i):
        slot = i % _N_BUF
        pltpu.make_async_copy(
            adj_ref.at[pl.ds(i * _CHUNK, _CHUNK), :],
            bufs_ref.at[slot], sems_ref.at[slot]).start()

    for i in range(_N_BUF):
        start(i)

    def mm(a, b):
        return jax.lax.dot_general(
            a, b, dimension_numbers=(((1,), (0,)), ((), ())),
            preferred_element_type=jnp.float32)

    xl = xl_ref[...]
    n_vars = _N_L // 2
    flipped = jnp.concatenate([xl[n_vars:], xl[:n_vars]], axis=0)
    gpart_ref[...] = mm(flipped, wflip_ref[...]) + mm(xl, whh_ref[...]) \
        + bias_ref[...]
    acc_ref[...] = jnp.zeros_like(acc_ref)

    def step(i, _):
        slot = jax.lax.rem(i, _N_BUF)
        pltpu.make_async_copy(
            adj_ref.at[pl.ds(i * _CHUNK, _CHUNK), :],
            bufs_ref.at[slot], sems_ref.at[slot]).wait()
        acc_ref[...] += jax.lax.dot_general(
            bufs_ref[slot], xc_ref[pl.ds(i * _CHUNK, _CHUNK), :],
            dimension_numbers=(((0,), (0,)), ((), ())),
            preferred_element_type=jnp.float32)

        @pl.when(i + _N_BUF < _N_CHUNKS)
        def _refill():
            nxt = i + _N_BUF
            pltpu.make_async_copy(
                adj_ref.at[pl.ds(nxt * _CHUNK, _CHUNK), :],
                bufs_ref.at[slot], sems_ref.at[slot]).start()
        return _

    jax.lax.fori_loop(0, _N_CHUNKS, step, 0)

    gates = gpart_ref[...] + mm(acc_ref[...], wmsg_ref[...])
    i_g = jax.nn.sigmoid(gates[:, :_D])
    f_g = jax.nn.sigmoid(gates[:, _D:2 * _D])
    g_g = jnp.tanh(gates[:, 2 * _D:3 * _D])
    o_g = jax.nn.sigmoid(gates[:, 3 * _D:])
    c = f_g * c0_ref[...] + i_g * g_g
    h_ref[...] = o_g * jnp.tanh(c)
    c_ref[...] = c


@functools.partial(jax.jit, static_argnames=())
def kernel(adj_t, x_c, hidden, l_batch, W_ih, W_hh, b_ih, b_hh):
    del l_batch
    x_l = hidden[0]
    c0 = hidden[1]
    wih_t = W_ih.T
    w_msg = wih_t[:_D]
    w_flip = wih_t[_D:]
    whh_t = W_hh.T
    bias = (b_ih + b_hh)[None, :]

    vmem = lambda: pl.BlockSpec(memory_space=pltpu.MemorySpace.VMEM)
    h, c = pl.pallas_call(
        _fused_body,
        in_specs=[
            pl.BlockSpec(memory_space=pltpu.MemorySpace.HBM),
            vmem(), vmem(), vmem(), vmem(), vmem(), vmem(), vmem(),
        ],
        out_specs=[vmem(), vmem()],
        out_shape=[jax.ShapeDtypeStruct((_N_L, _D), jnp.float32)] * 2,
        scratch_shapes=[
            pltpu.VMEM((_N_BUF, _CHUNK, _N_L), jnp.float32),
            pltpu.VMEM((_N_L, _D), jnp.float32),
            pltpu.VMEM((_N_L, 4 * _D), jnp.float32),
            pltpu.SemaphoreType.DMA((_N_BUF,)),
        ],
    )(adj_t, x_c, x_l, c0, w_msg, w_flip, whh_t, bias)
    return (h, c)
